# Initial kernel scaffold; baseline (speedup 1.0000x reference)
import jax
import jax.numpy as jnp
from jax.experimental import pallas as pl


def kernel(inputs, decoder_segment_ids, decoder_positions, gate_kernel, wi_0, wi_1, wo):
    raise NotImplementedError("write your pallas kernel here")


# speedup vs baseline: 1.3217x; 1.3217x over previous
import functools

import jax
import jax.numpy as jnp
from jax.experimental import pallas as pl
from jax.experimental.pallas import tpu as pltpu

D = 1024
F = 4096
NE = 8
TOPK = 2
RT = 128
FB = 1024
NF = F // FB


def _gemm_body(te_ref, xs_ref, wi0_ref, wi1_ref, wo_ref, out_ref):
    f = pl.program_id(0)
    t = pl.program_id(1)
    x = xs_ref[...]
    h0 = jnp.dot(x, wi0_ref[0], preferred_element_type=jnp.float32)
    h1 = jnp.dot(x, wi1_ref[0], preferred_element_type=jnp.float32)
    g = jax.nn.gelu(h0) * h1
    contrib = jnp.dot(g, wo_ref[0], preferred_element_type=jnp.float32)
    row = t * RT

    @pl.when(f == 0)
    def _():
        out_ref[pl.ds(row, RT), :] = contrib

    @pl.when(f != 0)
    def _():
        out_ref[pl.ds(row, RT), :] += contrib


def _grouped_gemm(xs, wi_0, wi_1, wo, tile_expert, nt):
    ntot = nt * RT
    grid_spec = pltpu.PrefetchScalarGridSpec(
        num_scalar_prefetch=1,
        grid=(NF, nt),
        in_specs=[
            pl.BlockSpec((RT, D), lambda f, t, te: (t, 0)),
            pl.BlockSpec((1, D, FB), lambda f, t, te: (te[t], 0, f)),
            pl.BlockSpec((1, D, FB), lambda f, t, te: (te[t], 0, f)),
            pl.BlockSpec((1, FB, D), lambda f, t, te: (te[t], f, 0)),
        ],
        out_specs=pl.BlockSpec((ntot, D), lambda f, t, te: (0, 0)),
    )
    return pl.pallas_call(
        _gemm_body,
        grid_spec=grid_spec,
        out_shape=jax.ShapeDtypeStruct((ntot, D), jnp.float32),
        compiler_params=pltpu.CompilerParams(
            dimension_semantics=("arbitrary", "arbitrary"),
        ),
    )(tile_expert, xs, wi_0, wi_1, wo)


def kernel(inputs, decoder_segment_ids, decoder_positions, gate_kernel,
           wi_0, wi_1, wo):
    b, s, d = inputs.shape
    t_tok = b * s
    n_disp = t_tok * TOPK
    nt = (n_disp + NE * (RT - 1) + RT - 1) // RT
    ntot = nt * RT

    xt = inputs.reshape(t_tok, d)

    logits = xt @ gate_kernel
    top_vals, top_idx = jax.lax.top_k(logits, TOPK)
    wts = jax.nn.softmax(top_vals.astype(jnp.float32), axis=-1)

    flat_e = top_idx.reshape(-1)
    one_hot = (flat_e[:, None] == jnp.arange(NE)[None, :]).astype(jnp.int32)
    counts = one_hot.sum(0)
    padded = ((counts + RT - 1) // RT) * RT
    group_start = jnp.concatenate(
        [jnp.zeros((1,), jnp.int32), jnp.cumsum(padded)[:-1].astype(jnp.int32)])
    rank = ((jnp.cumsum(one_hot, axis=0) - one_hot) * one_hot).sum(-1)
    pos = group_start[flat_e] + rank

    tile_start = group_start // RT
    tile_expert = (jnp.arange(nt, dtype=jnp.int32)[:, None]
                   >= tile_start[None, :]).sum(-1).astype(jnp.int32) - 1

    src = jnp.zeros((ntot,), jnp.int32).at[pos].set(
        jnp.arange(n_disp, dtype=jnp.int32) // TOPK)
    xs = xt[src]

    eo = _grouped_gemm(xs, wi_0, wi_1, wo, tile_expert, nt)

    pos2 = pos.reshape(t_tok, TOPK)
    out = (wts[:, :, None] * eo[pos2]).sum(1)
    return out.reshape(b, s, d)


# speedup vs baseline: 1.4163x; 1.0716x over previous
import functools

import jax
import jax.numpy as jnp
from jax.experimental import pallas as pl
from jax.experimental.pallas import tpu as pltpu

D = 1024
F = 4096
NE = 8
TOPK = 2
RT = 128
FB = 1024
NF = F // FB


def _router_body(x_ref, g_ref, idx_ref, wts_ref):
    x = x_ref[...]
    g = g_ref[...]
    logits = jnp.dot(x, g, preferred_element_type=jnp.float32)
    t = logits.shape[0]
    cols = jax.lax.broadcasted_iota(jnp.int32, (t, NE), 1)
    m1 = jnp.max(logits, axis=1)
    i1 = jnp.min(jnp.where(logits == m1[:, None], cols, NE), axis=1)
    masked = jnp.where(cols == i1[:, None], -jnp.inf, logits)
    m2 = jnp.max(masked, axis=1)
    i2 = jnp.min(jnp.where(masked == m2[:, None], cols, NE), axis=1)
    e2 = jnp.exp(m2 - m1)
    w1 = 1.0 / (1.0 + e2)
    w2 = e2 / (1.0 + e2)
    idx_ref[...] = jnp.stack([i1, i2])
    wts_ref[...] = jnp.stack([w1, w2])


def _router(xt, gate_kernel):
    t = xt.shape[0]
    return pl.pallas_call(
        _router_body,
        out_shape=(jax.ShapeDtypeStruct((TOPK, t), jnp.int32),
                   jax.ShapeDtypeStruct((TOPK, t), jnp.float32)),
    )(xt, gate_kernel)


def _gemm_body(te_ref, xs_ref, wi0_ref, wi1_ref, wo_ref, out_ref):
    f = pl.program_id(0)
    t = pl.program_id(1)
    x = xs_ref[...]
    h0 = jnp.dot(x, wi0_ref[0], preferred_element_type=jnp.float32)
    h1 = jnp.dot(x, wi1_ref[0], preferred_element_type=jnp.float32)
    g = jax.nn.gelu(h0) * h1
    contrib = jnp.dot(g, wo_ref[0], preferred_element_type=jnp.float32)
    row = t * RT

    @pl.when(f == 0)
    def _():
        out_ref[pl.ds(row, RT), :] = contrib

    @pl.when(f != 0)
    def _():
        out_ref[pl.ds(row, RT), :] += contrib


def _grouped_gemm(xs, wi_0, wi_1, wo, tile_expert, nt):
    ntot = nt * RT
    grid_spec = pltpu.PrefetchScalarGridSpec(
        num_scalar_prefetch=1,
        grid=(NF, nt),
        in_specs=[
            pl.BlockSpec((RT, D), lambda f, t, te: (t, 0)),
            pl.BlockSpec((1, D, FB), lambda f, t, te: (te[t], 0, f)),
            pl.BlockSpec((1, D, FB), lambda f, t, te: (te[t], 0, f)),
            pl.BlockSpec((1, FB, D), lambda f, t, te: (te[t], f, 0)),
        ],
        out_specs=pl.BlockSpec((ntot, D), lambda f, t, te: (0, 0)),
    )
    return pl.pallas_call(
        _gemm_body,
        grid_spec=grid_spec,
        out_shape=jax.ShapeDtypeStruct((ntot, D), jnp.float32),
        compiler_params=pltpu.CompilerParams(
            dimension_semantics=("arbitrary", "arbitrary"),
        ),
    )(tile_expert, xs, wi_0, wi_1, wo)


def kernel(inputs, decoder_segment_ids, decoder_positions, gate_kernel,
           wi_0, wi_1, wo):
    b, s, d = inputs.shape
    t_tok = b * s
    n_disp = t_tok * TOPK
    nt = (n_disp + NE * (RT - 1) + RT - 1) // RT
    ntot = nt * RT

    xt = inputs.reshape(t_tok, d)

    idx_kt, wts_kt = _router(xt, gate_kernel)

    flat_e = idx_kt.reshape(-1)
    one_hot = (flat_e[:, None] == jnp.arange(NE)[None, :]).astype(jnp.int32)
    counts = one_hot.sum(0)
    padded = ((counts + RT - 1) // RT) * RT
    group_start = jnp.concatenate(
        [jnp.zeros((1,), jnp.int32), jnp.cumsum(padded)[:-1].astype(jnp.int32)])
    rank = ((jnp.cumsum(one_hot, axis=0) - one_hot) * one_hot).sum(-1)
    pos = group_start[flat_e] + rank

    tile_start = group_start // RT
    tile_expert = (jnp.arange(nt, dtype=jnp.int32)[:, None]
                   >= tile_start[None, :]).sum(-1).astype(jnp.int32) - 1

    src = jnp.zeros((ntot,), jnp.int32).at[pos].set(
        jnp.arange(n_disp, dtype=jnp.int32) % t_tok)
    xs = xt[src]

    eo = _grouped_gemm(xs, wi_0, wi_1, wo, tile_expert, nt)

    pos_kt = pos.reshape(TOPK, t_tok)
    out = (wts_kt[0][:, None] * eo[pos_kt[0]]
           + wts_kt[1][:, None] * eo[pos_kt[1]])
    return out.reshape(b, s, d)
